# Initial kernel scaffold; baseline (speedup 1.0000x reference)
import jax
import jax.numpy as jnp
from jax.experimental import pallas as pl


def kernel(pos, h, edge_attr, edge_index, node_time_emb, edge_time_emb, params):
    raise NotImplementedError("write your pallas kernel here")


# speedup vs baseline: 5.3327x; 5.3327x over previous
import functools

import jax
import jax.numpy as jnp
import numpy as np
from jax import lax
from jax.experimental import pallas as pl
from jax.experimental.pallas import tpu as pltpu
from jax.experimental.pallas import tpu_sc as plsc

N = 10000
E = 320000
D = 128
ED = 16
DD = 16
T = 128
H = 8
C = 16

NC = 2
NS = 16
NW = NC * NS
CH = 128
KCH = 79
E_pad = NW * CH * KCH

BN = 2000
BE = 2048
GN = N // BN
GE = E_pad // BE
LAST_IN_BLK = (E + BE - 1) // BE - 1

_f32 = jnp.float32


def _silu(x):
    return x * jax.nn.sigmoid(x)


def _ln(x, eps=1e-6):
    m = jnp.mean(x, axis=-1, keepdims=True)
    v = jnp.var(x, axis=-1, keepdims=True)
    return (x - m) / jnp.sqrt(v + eps)


def _full(shape):
    return pl.BlockSpec(shape, lambda i: tuple(0 for _ in shape))


def _rb(b, w):
    return pl.BlockSpec((b, w), lambda i: (i, 0))


def _rbc(b, w, maxi):
    return pl.BlockSpec((b, w), lambda i: (jnp.minimum(i, maxi), 0))


def _node1_body(h_ref, nte_ref, pos16_ref, wntm_ref, bntm_ref, wq_ref, bq_ref,
                wk_ref, bk_ref, wv_ref, bv_ref,
                qp_ref, kp_ref, v_ref, nrest_ref):
    h = h_ref[...]
    nmod = _silu(nte_ref[...]) @ wntm_ref[...] + bntm_ref[...]
    sh, sc = nmod[:, 0:D], nmod[:, D:2 * D]
    hm = _ln(h) * (1.0 + sc) + sh
    q = hm @ wq_ref[...] + bq_ref[...]
    k = hm @ wk_ref[...] + bk_ref[...]
    v = hm @ wv_ref[...] + bv_ref[...]
    p16 = pos16_ref[...]
    pad = jnp.zeros((BN, 2 * D - (D + 16)), _f32)
    qp_ref[...] = jnp.concatenate([q, p16, pad], axis=1)
    kp_ref[...] = jnp.concatenate([k, p16, pad], axis=1)
    v_ref[...] = v
    nrest_ref[...] = nmod[:, 2 * D:]


def _edge1_body(qpd_ref, kps_ref, ea_ref, ete_ref, wee_ref, bee_ref,
                wetm_ref, betm_ref, we0_ref, we1_ref, gm_ref, gs_ref, csc_ref,
                ex16_ref, e1_ref, dc_ref, erest_ref):
    blk = pl.program_id(0)
    rows = blk * BE + lax.broadcasted_iota(jnp.int32, (BE, 1), 0)
    valid = (rows < E).astype(_f32)

    qpd = qpd_ref[...]
    kps = kps_ref[...]
    cd0 = kps[:, D:D + 16] - qpd[:, D:D + 16]
    radial = jnp.sum(cd0 * cd0, axis=1, keepdims=True)
    gm = gm_ref[...]
    gs = gs_ref[...]
    a = (2.0 * 3.14159) ** 0.5
    gauss = jnp.exp(-0.5 * ((radial - gm) / gs) ** 2) / (a * gs)
    col0 = lax.broadcasted_iota(jnp.int32, (BE, 16), 1) == 0
    distance = jnp.where(col0, radial, gauss)
    nrm = jnp.sqrt(radial)
    cdn = cd0 / jnp.maximum(nrm, 1e-8) * csc_ref[0, 0]
    dc_ref[...] = jnp.concatenate([distance, cdn], axis=1) * valid

    ea2 = jnp.concatenate([distance, ea_ref[...]], axis=1) @ wee_ref[...] + bee_ref[...]
    emod = _silu(ete_ref[...]) @ wetm_ref[...] + betm_ref[...]
    em = _ln(ea2) * (1.0 + emod[:, ED:2 * ED]) + emod[:, 0:ED]
    e0 = jnp.tanh(em @ we0_ref[...])
    e1 = jnp.tanh(em @ we1_ref[...])
    e1_ref[...] = e1
    erest_ref[...] = emod[:, 2 * ED:]

    t = qpd[:, :D] * kps[:, :D] * e0
    hsel = (lax.broadcasted_iota(jnp.int32, (D, H), 0) // C
            == lax.broadcasted_iota(jnp.int32, (D, H), 1)).astype(_f32)
    alpha = (t @ hsel) * (1.0 / np.sqrt(C).astype(np.float32))
    ex = jnp.exp(alpha) * valid
    ex16_ref[...] = jnp.concatenate([ex, jnp.zeros((BE, 8), _f32)], axis=1)


def _edge2_body(vs_ref, e1_ref, ex16_ref, msg_ref):
    rep = (lax.broadcasted_iota(jnp.int32, (16, D), 1) // C
           == lax.broadcasted_iota(jnp.int32, (16, D), 0)).astype(_f32)
    msg_ref[...] = vs_ref[...] * e1_ref[...] * (ex16_ref[...] @ rep)


def _node2_body(m0_ref, m1_ref, d0_ref, d1_ref, h_ref, nrest_ref,
                wproj_ref, bproj_ref,
                wff1_ref, bff1_ref, wff2_ref, bff2_ref, hout_ref, tbl_ref):
    invd = 1.0 / (d0_ref[...] + d1_ref[...] + 1e-16)
    rep = (lax.broadcasted_iota(jnp.int32, (16, D), 1) // C
           == lax.broadcasted_iota(jnp.int32, (16, D), 0)).astype(_f32)
    agg = (m0_ref[...] + m1_ref[...]) * (invd @ rep)
    hp = agg @ wproj_ref[...] + bproj_ref[...]
    nrest = nrest_ref[...]
    h_node = h_ref[...] + nrest[:, 0:D] * hp
    x = _ln(h_node) * (1.0 + nrest[:, 2 * D:3 * D]) + nrest[:, D:2 * D]
    ffn = _silu(x @ wff1_ref[...] + bff1_ref[...]) @ wff2_ref[...] + bff2_ref[...]
    h_out = h_node + nrest[:, 3 * D:] * ffn
    hout_ref[...] = h_out
    tbl_ref[...] = jnp.concatenate([hp, h_out], axis=1)


def _edge3_body(tbls_ref, tbld_ref, ea_ref, ete_ref, erest_ref, dc_ref,
                wn2e_ref, bn2e_ref, wff3_ref, bff3_ref, wff4_ref, bff4_ref,
                wtm_ref, btm_ref, wi1_ref, wi2_ref, wi3_ref, wi4_ref, binp_ref,
                wcm1_ref, bcm1_ref, wcm2_ref,
                heout_ref, trans_ref):
    tbls = tbls_ref[...]
    tbld = tbld_ref[...]
    erest = erest_ref[...]
    e_g_msa = erest[:, 0:ED]
    e_sh_mlp = erest[:, ED:2 * ED]
    e_sc_mlp = erest[:, 2 * ED:3 * ED]
    e_g_mlp = erest[:, 3 * ED:4 * ED]

    h_edgeM = (tbls[:, :D] + tbld[:, :D]) @ wn2e_ref[...] + bn2e_ref[...]
    h_edge = ea_ref[...] + e_g_msa * h_edgeM
    e_out = _ln(h_edge) * (1.0 + e_sc_mlp) + e_sh_mlp
    ffe = _silu(e_out @ wff3_ref[...] + bff3_ref[...]) @ wff4_ref[...] + bff4_ref[...]
    h_edge_out = h_edge + e_g_mlp * ffe
    heout_ref[...] = h_edge_out

    tm = _silu(ete_ref[...]) @ wtm_ref[...] + btm_ref[...]
    dc = dc_ref[...]
    hin = (tbls[:, D:] @ wi1_ref[...] + tbld[:, D:] @ wi2_ref[...]
           + h_edge_out @ wi3_ref[...] + dc[:, :16] @ wi4_ref[...] + binp_ref[...])
    inv = _ln(hin) * (1.0 + tm[:, D:]) + tm[:, 0:D]
    inv = jnp.tanh(_silu(inv @ wcm1_ref[...] + bcm1_ref[...]) @ wcm2_ref[...])
    trans_ref[...] = dc[:, 16:32] * inv


def _pos_body(pos16_ref, t0_ref, t1_ref, out_ref):
    out_ref[...] = pos16_ref[...] + t0_ref[...] + t1_ref[...]


_MESH = plsc.VectorSubcoreMesh(core_axis_name="c", subcore_axis_name="s",
                               num_cores=NC, num_subcores=NS)


def _make_gather2(wa, wb):
    @functools.partial(
        pl.kernel,
        out_type=(jax.ShapeDtypeStruct((E_pad, wa), _f32),
                  jax.ShapeDtypeStruct((E_pad, wb), _f32)),
        mesh=_MESH,
        scratch_types=[
            pltpu.VMEM((CH,), jnp.int32),
            pltpu.VMEM((CH,), jnp.int32),
            pltpu.VMEM((CH, wa), _f32),
            pltpu.VMEM((CH, wb), _f32),
            pltpu.SemaphoreType.DMA,
            pltpu.SemaphoreType.DMA,
        ],
    )
    def gather2(taba, tabb, idxa, idxb, outa, outb,
                idxa_v, idxb_v, bufa, bufb, sema, semb):
        wid = lax.axis_index("s") * NC + lax.axis_index("c")
        base = wid * KCH

        def chunk(i, carry):
            ebase = (base + i) * CH
            pltpu.sync_copy(idxa.at[pl.ds(ebase, CH)], idxa_v)
            pltpu.sync_copy(idxb.at[pl.ds(ebase, CH)], idxb_v)
            cpa = pltpu.async_copy(taba.at[idxa_v], bufa, sema)
            cpb = pltpu.async_copy(tabb.at[idxb_v], bufb, semb)
            cpa.wait()
            cpb.wait()
            pltpu.sync_copy(bufa, outa.at[pl.ds(ebase, CH)])
            pltpu.sync_copy(bufb, outb.at[pl.ds(ebase, CH)])
            return carry

        lax.fori_loop(0, KCH, chunk, 0)

    return gather2


N_sub = ((N + NS * 8 - 1) // (NS * 8)) * 8
N_pad = NS * N_sub


def _make_scatter(w):
    sizes = []
    off = 0
    rows_per_sub = N_sub
    while off < rows_per_sub:
        sz = min(CH, rows_per_sub - off)
        sizes.append((off, sz))
        off += sz

    @functools.partial(
        pl.kernel,
        out_type=jax.ShapeDtypeStruct((NC * N_pad, w), _f32),
        mesh=_MESH,
        scratch_types=[
            pltpu.VMEM((1, CH), jnp.int32),
            pltpu.VMEM((CH, w), _f32),
            pltpu.VMEM((8, w), _f32),
            pltpu.VMEM_SHARED((N_pad, w), _f32),
            pltpu.SemaphoreType.DMA,
        ],
    )
    def scatter(vals, idx2, out, idx_v, val_v, zbuf, acc, sem):
        cid = lax.axis_index("c")
        sid = lax.axis_index("s")
        wid = sid * NC + cid
        base = wid * KCH
        rbase = sid * rows_per_sub

        for r in range(8):
            for c2 in range(w // 16):
                zbuf[r, pl.ds(c2 * 16, 16)] = jnp.zeros((16,), _f32)
        def zfill(g, carry):
            pltpu.sync_copy(zbuf, acc.at[pl.ds(rbase + g * 8, 8)])
            return carry

        lax.fori_loop(0, rows_per_sub // 8, zfill, 0)
        plsc.subcore_barrier()

        def chunk(i, carry):
            ebase = (base + i) * CH
            pltpu.sync_copy(idx2.at[pl.ds(base + i, 1)], idx_v)
            pltpu.sync_copy(vals.at[pl.ds(ebase, CH)], val_v)
            pltpu.async_copy(val_v, acc.at[idx_v.at[0]], sem, add=True).wait()
            return carry

        lax.fori_loop(0, KCH, chunk, 0)
        plsc.subcore_barrier()
        for off, sz in sizes:
            pltpu.sync_copy(acc.at[pl.ds(rbase + off, sz)],
                            out.at[pl.ds(cid * N_pad + rbase + off, sz)])

    return scatter


def _make_gather1(w):
    @functools.partial(
        pl.kernel,
        out_type=jax.ShapeDtypeStruct((E_pad, w), _f32),
        mesh=_MESH,
        scratch_types=[
            pltpu.VMEM((CH,), jnp.int32),
            pltpu.VMEM((CH, w), _f32),
            pltpu.SemaphoreType.DMA,
        ],
    )
    def gather1(tab, idx, out, idx_v, buf, sem):
        wid = lax.axis_index("s") * NC + lax.axis_index("c")
        base = wid * KCH

        def chunk(i, carry):
            ebase = (base + i) * CH
            pltpu.sync_copy(idx.at[pl.ds(ebase, CH)], idx_v)
            pltpu.async_copy(tab.at[idx_v], buf, sem).wait()
            pltpu.sync_copy(buf, out.at[pl.ds(ebase, CH)])
            return carry

        lax.fori_loop(0, KCH, chunk, 0)

    return gather1


_gather_qk = _make_gather2(2 * D, 2 * D)
_gather_v = _make_gather1(D)
_gather_tbl = _make_gather2(2 * D, 2 * D)
_scatter16 = _make_scatter(16)
_scatter128 = _make_scatter(D)


N2 = 10240
BES = 512
GES = E_pad // BES


def _seg_body(idx_ref, vals_ref, out_ref):
    i = pl.program_id(0)

    @pl.when(i == 0)
    def _zero():
        out_ref[...] = jnp.zeros_like(out_ref)

    ids = idx_ref[...].reshape(1, BES)
    oh = (lax.broadcasted_iota(jnp.int32, (N2, BES), 0) == ids).astype(_f32)
    out_ref[...] += jnp.dot(oh, vals_ref[...],
                            preferred_element_type=_f32)


def _tc_scatter(vals, idx2):
    w = vals.shape[1]
    out = pl.pallas_call(
        _seg_body,
        grid=(GES,),
        in_specs=[pl.BlockSpec((BES, 1), lambda i: (i, 0)),
                  pl.BlockSpec((BES, w), lambda i: (i, 0))],
        out_specs=pl.BlockSpec((N2, w), lambda i: (0, 0)),
        out_shape=jax.ShapeDtypeStruct((N2, w), _f32),
    )(idx2.reshape(E_pad, 1), vals)
    return jnp.concatenate([out[:N_pad], jnp.zeros((N_pad, w), _f32)])


def kernel(pos, h, edge_attr, edge_index, node_time_emb, edge_time_emb, params):
    g2, g2t, g1, s16, s128 = (_gather_qk, _gather_tbl, _gather_v,
                              _tc_scatter, _tc_scatter)
    p = params
    src = jnp.pad(edge_index[0], (0, E_pad - E))
    dst = jnp.pad(edge_index[1], (0, E_pad - E))
    src2 = src.reshape(-1, CH)
    dst2 = dst.reshape(-1, CH)
    pos16 = jnp.pad(pos, ((0, 0), (0, 13)))

    r2 = lambda b: b.reshape(1, -1)
    wntm_t = p['W_ntm'].T
    wee_t = p['W_edge_emb'].T
    wetm_t = p['W_etm'].T
    wi = p['W_inp']
    csc = p['coors_scale'][0]

    qp_tab, kp_tab, v_tab, nrest = pl.pallas_call(
        _node1_body,
        grid=(GN,),
        in_specs=[_rb(BN, D), _rb(BN, T), _rb(BN, 16),
                  _full((D, 6 * D)), _full((1, 6 * D)),
                  _full((D, D)), _full((1, D)), _full((D, D)), _full((1, D)),
                  _full((D, D)), _full((1, D))],
        out_specs=[_rb(BN, 2 * D), _rb(BN, 2 * D), _rb(BN, D), _rb(BN, 4 * D)],
        out_shape=[jax.ShapeDtypeStruct((N, 2 * D), _f32),
                   jax.ShapeDtypeStruct((N, 2 * D), _f32),
                   jax.ShapeDtypeStruct((N, D), _f32),
                   jax.ShapeDtypeStruct((N, 4 * D), _f32)],
    )(h, node_time_emb, pos16, wntm_t, r2(p['b_ntm']),
      p['Wq'].T, r2(p['bq']), p['Wk'].T, r2(p['bk']), p['Wv'].T, r2(p['bv']))

    qpd, kps = g2(qp_tab, kp_tab, dst, src)

    gm16 = jnp.concatenate([jnp.zeros((1,), _f32), p['g_means']]).reshape(1, 16)
    gs16 = jnp.concatenate([jnp.ones((1,), _f32),
                            jnp.abs(p['g_stds']) + 1e-5]).reshape(1, 16)
    ex16, e1, dist_cdn, erest = pl.pallas_call(
        _edge1_body,
        grid=(GE,),
        in_specs=[_rb(BE, 2 * D), _rb(BE, 2 * D),
                  _rbc(BE, ED, LAST_IN_BLK), _rbc(BE, T, LAST_IN_BLK),
                  _full((DD + ED, ED)), _full((1, ED)),
                  _full((T, 6 * ED)), _full((1, 6 * ED)),
                  _full((ED, D)), _full((ED, D)),
                  _full((1, 16)), _full((1, 16)), _full((1, 1))],
        out_specs=[_rb(BE, 16), _rb(BE, D), _rb(BE, 32), _rb(BE, 4 * ED)],
        out_shape=[jax.ShapeDtypeStruct((E_pad, 16), _f32),
                   jax.ShapeDtypeStruct((E_pad, D), _f32),
                   jax.ShapeDtypeStruct((E_pad, 32), _f32),
                   jax.ShapeDtypeStruct((E_pad, 4 * ED), _f32)],
    )(qpd, kps, edge_attr, edge_time_emb, wee_t, r2(p['b_edge_emb']),
      wetm_t, r2(p['b_etm']), p['We0'].T, p['We1'].T, gm16, gs16,
      csc.reshape(1, 1))

    den_parts = s16(ex16, dst2)

    vs = g1(v_tab, src)

    msg = pl.pallas_call(
        _edge2_body,
        grid=(GE,),
        in_specs=[_rb(BE, D), _rb(BE, D), _rb(BE, 16)],
        out_specs=_rb(BE, D),
        out_shape=jax.ShapeDtypeStruct((E_pad, D), _f32),
    )(vs, e1, ex16)

    msg_parts = s128(msg, dst2)

    h_out, tbl = pl.pallas_call(
        _node2_body,
        grid=(GN,),
        in_specs=[_rb(BN, D), _rb(BN, D), _rb(BN, 16), _rb(BN, 16),
                  _rb(BN, D), _rb(BN, 4 * D),
                  _full((D, D)), _full((1, D)),
                  _full((D, 2 * D)), _full((1, 2 * D)),
                  _full((2 * D, D)), _full((1, D))],
        out_specs=[_rb(BN, D), _rb(BN, 2 * D)],
        out_shape=[jax.ShapeDtypeStruct((N, D), _f32),
                   jax.ShapeDtypeStruct((N, 2 * D), _f32)],
    )(msg_parts[:N_pad], msg_parts[N_pad:], den_parts[:N_pad], den_parts[N_pad:], h, nrest,
      p['Wproj'].T, r2(p['bproj']),
      p['W_ff1'].T, r2(p['b_ff1']), p['W_ff2'].T, r2(p['b_ff2']))

    tbls, tbld = g2t(tbl, tbl, src, dst)

    h_edge_out, trans16 = pl.pallas_call(
        _edge3_body,
        grid=(GE,),
        in_specs=[_rb(BE, 2 * D), _rb(BE, 2 * D),
                  _rbc(BE, ED, LAST_IN_BLK), _rbc(BE, T, LAST_IN_BLK),
                  _rb(BE, 4 * ED), _rb(BE, 32),
                  _full((D, ED)), _full((1, ED)),
                  _full((ED, 2 * ED)), _full((1, 2 * ED)),
                  _full((2 * ED, ED)), _full((1, ED)),
                  _full((T, 2 * D)), _full((1, 2 * D)),
                  _full((D, D)), _full((D, D)), _full((ED, D)), _full((ED, D)),
                  _full((1, D)),
                  _full((D, D)), _full((1, D)), _full((D, 1))],
        out_specs=[_rb(BE, ED), _rb(BE, 16)],
        out_shape=[jax.ShapeDtypeStruct((E_pad, ED), _f32),
                   jax.ShapeDtypeStruct((E_pad, 16), _f32)],
    )(tbls, tbld, edge_attr, edge_time_emb, erest, dist_cdn,
      p['W_n2e'].T, r2(p['b_n2e']),
      p['W_ff3'].T, r2(p['b_ff3']), p['W_ff4'].T, r2(p['b_ff4']),
      p['W_tmlp'].T, r2(p['b_tmlp']),
      wi[:, :D].T, wi[:, D:2 * D].T, wi[:, 2 * D:2 * D + ED].T,
      wi[:, 2 * D + ED:].T, r2(p['b_inp']),
      p['W_cm1'].T, r2(p['b_cm1']), p['W_cm2'].T)

    trans_parts = s16(trans16, src2)

    pos16_out = pl.pallas_call(
        _pos_body,
        grid=(GN,),
        in_specs=[_rb(BN, 16), _rb(BN, 16), _rb(BN, 16)],
        out_specs=_rb(BN, 16),
        out_shape=jax.ShapeDtypeStruct((N, 16), _f32),
    )(pos16, trans_parts[:N_pad], trans_parts[N_pad:])

    return h_out, h_edge_out[:E], pos16_out[:, :3]
